# baseline (speedup 1.0000x reference)
import functools

import jax
import jax.numpy as jnp
from jax.experimental import pallas as pl
from jax.experimental.pallas import tpu as pltpu

VMEM_LIMIT = 64 * 1024 * 1024


def _gcn_kernel(x_ref, adj_lo_ref, adj_hi_ref, adj_q2_ref, adj_q3_ref,
                w1_ref, w2_ref, b1_ref,
                b2_ref, out_ref, s1_ref, acc_ref, *, n_steps):
    t = pl.program_id(0)

    @pl.when(t == 0)
    def _():
        s1_ref[...] = jnp.dot(
            x_ref[...].astype(jnp.bfloat16), w1_ref[...],
            preferred_element_type=jnp.float32).astype(jnp.bfloat16)

    @pl.when(t > 0)
    def _():
        def partial(adj_slab_ref):
            a = adj_slab_ref[...].astype(jnp.bfloat16)
            h1 = jnp.dot(a, s1_ref[...], preferred_element_type=jnp.float32)
            h1 = jnp.maximum(h1 + b1_ref[...], 0.0).astype(jnp.bfloat16)
            s2_t = jnp.dot(
                h1, w2_ref[...],
                preferred_element_type=jnp.float32).astype(jnp.bfloat16)
            dn = (((0,), (0,)), ((), ()))
            return jax.lax.dot_general(
                s2_t, a, dn, preferred_element_type=jnp.float32)

        pm = (partial(adj_lo_ref) + partial(adj_hi_ref)
              + partial(adj_q2_ref) + partial(adj_q3_ref))
        @pl.when(t == 1)
        def _():
            acc_ref[...] = pm
        @pl.when(t > 1)
        def _():
            acc_ref[...] += pm

        @pl.when(t == n_steps - 1)
        def _():
            out_ref[...] = acc_ref[...].T + b2_ref[...]


def kernel(feature, adj, w1, b1, w2, b2):
    n, nfeat = feature.shape
    nhid1 = w1.shape[1]
    nhid2 = w2.shape[1]

    w1_bf = w1.astype(jnp.bfloat16)
    w2_bf = w2.astype(jnp.bfloat16)
    b1_2d = b1.reshape(1, nhid1).astype(jnp.float32)
    b2_2d = b2.reshape(1, nhid2).astype(jnp.float32)

    slab = 256
    q_slabs = n // (4 * slab)
    n_steps = q_slabs + 1

    body = functools.partial(_gcn_kernel, n_steps=n_steps)
    def qmap(q):
        return lambda t: (q * q_slabs + jnp.maximum(t - 1, 0), 0)
    out = pl.pallas_call(
        body,
        out_shape=jax.ShapeDtypeStruct((n, nhid2), jnp.float32),
        grid=(n_steps,),
        in_specs=[
            pl.BlockSpec((n, nfeat), lambda t: (0, 0)),
            pl.BlockSpec((slab, n), qmap(0)),
            pl.BlockSpec((slab, n), qmap(1)),
            pl.BlockSpec((slab, n), qmap(2)),
            pl.BlockSpec((slab, n), qmap(3)),
            pl.BlockSpec((nfeat, nhid1), lambda t: (0, 0)),
            pl.BlockSpec((nhid1, nhid2), lambda t: (0, 0)),
            pl.BlockSpec((1, nhid1), lambda t: (0, 0)),
            pl.BlockSpec((1, nhid2), lambda t: (0, 0)),
        ],
        out_specs=pl.BlockSpec((n, nhid2), lambda t: (0, 0)),
        scratch_shapes=[
            pltpu.VMEM((n, nhid1), jnp.bfloat16),
            pltpu.VMEM((nhid2, n), jnp.float32),
        ],
        compiler_params=pltpu.CompilerParams(
            dimension_semantics=("arbitrary",),
            vmem_limit_bytes=VMEM_LIMIT),
    )(feature, adj, adj, adj, adj, w1_bf, w2_bf, b1_2d, b2_2d)
    return out
